# Initial kernel scaffold; baseline (speedup 1.0000x reference)
import jax
import jax.numpy as jnp
from jax.experimental import pallas as pl


def kernel(x, vals, rows, cols):
    raise NotImplementedError("write your pallas kernel here")


# speedup vs baseline: 6.5356x; 6.5356x over previous
import functools

import jax
import jax.numpy as jnp
from jax import lax
from jax.experimental import pallas as pl
from jax.experimental.pallas import tpu as pltpu
from jax.experimental.pallas import tpu_sc as plsc

IN_F = 4096
COLS = 256
HALF = 128
NSUB = 16
NCORE = 2
CHUNK = 128
LANES = 16
VPR = HALF // LANES


def _sc_spmm(x0, x1, rows, cols, vals):
    nnz_pad = vals.shape[0]
    nnz_per_sub = nnz_pad // NSUB
    n_chunks = nnz_per_sub // CHUNK
    rows_per_sub = IN_F // NSUB

    mesh = plsc.VectorSubcoreMesh(core_axis_name="c", subcore_axis_name="s",
                                  num_cores=NCORE, num_subcores=NSUB)

    @functools.partial(
        pl.kernel,
        out_type=(jax.ShapeDtypeStruct((IN_F, HALF), jnp.float32),
                  jax.ShapeDtypeStruct((IN_F, HALF), jnp.float32)),
        mesh=mesh,
        scratch_types=[
            pltpu.VMEM((CHUNK,), jnp.int32),
            pltpu.VMEM((CHUNK,), jnp.int32),
            pltpu.VMEM((CHUNK,), jnp.float32),
            pltpu.VMEM((CHUNK, HALF), jnp.float32),
            pltpu.VMEM_SHARED((IN_F, HALF), jnp.float32),
            pltpu.SemaphoreType.DMA,
            pltpu.SemaphoreType.DMA,
        ],
    )
    def k(x0_hbm, x1_hbm, rows_hbm, cols_hbm, vals_hbm, out0_hbm, out1_hbm,
          cidx_v, ridx_v, vals_v, gbuf, acc_sh, sem, gsem):
        cid = lax.axis_index("c")
        sid = lax.axis_index("s")
        base = sid * nnz_per_sub
        r0 = sid * rows_per_sub

        def zbody(i, _):
            for j in range(VPR):
                gbuf[i, pl.ds(j * LANES, LANES)] = jnp.zeros((LANES,),
                                                             jnp.float32)
            return 0
        lax.fori_loop(0, CHUNK, zbody, 0)
        pltpu.sync_copy(gbuf, acc_sh.at[pl.ds(r0, CHUNK)])
        pltpu.sync_copy(gbuf, acc_sh.at[pl.ds(r0 + CHUNK, CHUNK)])
        plsc.subcore_barrier()

        def chunk_body(g, _):
            off = base + g * CHUNK
            cp1 = pltpu.async_copy(cols_hbm.at[pl.ds(off, CHUNK)], cidx_v, sem)
            cp2 = pltpu.async_copy(rows_hbm.at[pl.ds(off, CHUNK)], ridx_v, sem)
            cp3 = pltpu.async_copy(vals_hbm.at[pl.ds(off, CHUNK)], vals_v, sem)
            cp1.wait()
            cp2.wait()
            cp3.wait()

            @pl.when(cid == 0)
            def _():
                pltpu.async_copy(x0_hbm.at[cidx_v], gbuf, gsem).wait()

            @pl.when(cid == 1)
            def _():
                pltpu.async_copy(x1_hbm.at[cidx_v], gbuf, gsem).wait()

            def mul_body(kk, _):
                vv16 = vals_v[pl.ds(kk * LANES, LANES)]
                for l in range(LANES):
                    vv = jnp.full((LANES,), vv16[l], jnp.float32)
                    row = kk * LANES + l
                    for j in range(VPR):
                        sl = pl.ds(j * LANES, LANES)
                        gbuf[row, sl] = gbuf[row, sl] * vv
                return 0
            lax.fori_loop(0, CHUNK // LANES, mul_body, 0)

            pltpu.sync_copy(gbuf, acc_sh.at[ridx_v], add=True)
            return 0
        lax.fori_loop(0, n_chunks, chunk_body, 0)
        plsc.subcore_barrier()

        @pl.when(cid == 0)
        def _():
            pltpu.sync_copy(acc_sh.at[pl.ds(r0, rows_per_sub)],
                            out0_hbm.at[pl.ds(r0, rows_per_sub)])

        @pl.when(cid == 1)
        def _():
            pltpu.sync_copy(acc_sh.at[pl.ds(r0, rows_per_sub)],
                            out1_hbm.at[pl.ds(r0, rows_per_sub)])

    return k(x0, x1, rows, cols, vals)


def kernel(x, vals, rows, cols):
    nnz = vals.shape[0]
    grp = NSUB * CHUNK
    nnz_pad = ((nnz + grp - 1) // grp) * grp
    pad = nnz_pad - nnz
    rows_p = jnp.pad(rows, (0, pad))
    cols_p = jnp.pad(cols, (0, pad))
    vals_p = jnp.pad(vals, (0, pad))
    x0 = x[:, :HALF]
    x1 = x[:, HALF:]
    out0, out1 = _sc_spmm(x0, x1, rows_p, cols_p, vals_p)
    return jnp.concatenate([out0, out1], axis=1)
